# baseline (speedup 1.0000x reference)
import jax
import jax.numpy as jnp
from jax.experimental import pallas as pl

N = 16384
K = 1000
F = 16
TN = 1024
G = N // TN


def _body(xa_ref, ca_ref, dist_ref, asg_ref, ine_ref):
    xa = xa_ref[...]
    ca = ca_ref[...]
    d2 = jax.lax.dot_general(
        xa, ca, (((1,), (1,)), ((), ())), preferred_element_type=jnp.float32
    )
    d2 = jnp.maximum(d2, 0.0)
    dist_ref[...] = jnp.sqrt(d2)
    m = jnp.min(d2, axis=1)
    iota = jax.lax.broadcasted_iota(jnp.int32, d2.shape, 1)
    idx = jnp.min(jnp.where(d2 == m[:, None], iota, K), axis=1)
    asg_ref[0, 0, :] = idx
    ine_ref[0, 0, :] = m


def kernel(data, centroids):
    x2 = jnp.sum(data * data, axis=1, keepdims=True)
    c2 = jnp.sum(centroids * centroids, axis=1, keepdims=True)
    ones_x = jnp.ones_like(x2)
    ones_c = jnp.ones_like(c2)
    xa = jnp.concatenate([-2.0 * data, ones_x, x2], axis=1)
    ca = jnp.concatenate([centroids, c2, ones_c], axis=1)

    distances, asg3, ine3 = pl.pallas_call(
        _body,
        grid=(G,),
        in_specs=[
            pl.BlockSpec((TN, F + 2), lambda i: (i, 0)),
            pl.BlockSpec((K, F + 2), lambda i: (0, 0)),
        ],
        out_specs=[
            pl.BlockSpec((TN, K), lambda i: (i, 0)),
            pl.BlockSpec((1, 1, TN), lambda i: (i, 0, 0)),
            pl.BlockSpec((1, 1, TN), lambda i: (i, 0, 0)),
        ],
        out_shape=[
            jax.ShapeDtypeStruct((N, K), jnp.float32),
            jax.ShapeDtypeStruct((G, 1, TN), jnp.int32),
            jax.ShapeDtypeStruct((G, 1, TN), jnp.float32),
        ],
    )(xa, ca)
    return distances, asg3.reshape(N), ine3.reshape(N)
